# Initial kernel scaffold; baseline (speedup 1.0000x reference)
import jax
import jax.numpy as jnp
from jax.experimental import pallas as pl


def kernel(idx, tok_emb, wpe):
    raise NotImplementedError("write your pallas kernel here")


# speedup vs baseline: 1.3324x; 1.3324x over previous
import functools

import jax
import jax.numpy as jnp
from jax import lax
from jax.experimental import pallas as pl
from jax.experimental.pallas import tpu as pltpu
from jax.experimental.pallas import tpu_sc as plsc

_VOCAB = 100000
_N_EMBD = 768
_BLOCK = 2048
_BATCH = 4

_NC = 2
_NS = 16
_NW = _NC * _NS
_P = _BLOCK // _NW
_Q = 16
_NQ = _P // _Q
_G = _N_EMBD // 16
_WAVE = 4


def _emb_body(idx_hbm, tok_hbm, wpe_hbm, out_hbm,
              idx_v, wpe0, wpe1, tok_v,
              sg0, sg1, sw0, sw1, ss0, ss1):
    wid = lax.axis_index("s") * _NC + lax.axis_index("c")
    pos_base = wid * _P
    wpe_b = (wpe0, wpe1)
    sg = (sg0, sg1)
    sw = (sw0, sw1)
    ss = (ss0, ss1)

    for b in range(_BATCH):
        pltpu.sync_copy(idx_hbm.at[pl.ds(b * _BLOCK + pos_base, _P)],
                        idx_v.at[pl.ds(b * _P, _P)])

    def issue_quarter(q):
        par = q % 2
        gh = [pltpu.async_copy(
                  tok_hbm.at[idx_v.at[pl.ds(b * _P + q * _Q, _Q)]],
                  tok_v.at[par * _BATCH + b], sg[par])
              for b in range(_BATCH)]
        wh = pltpu.async_copy(
            wpe_hbm.at[pl.ds(pos_base + q * _Q, _Q)], wpe_b[par], sw[par])
        return gh, wh

    handles = {0: issue_quarter(0)}
    store_h = {}

    for q in range(_NQ):
        par = q % 2
        if q + 1 < _NQ:
            if q - 1 >= 0:
                for h in store_h.pop(q - 1):
                    h.wait()
            handles[q + 1] = issue_quarter(q + 1)
        gh, wh = handles.pop(q)
        for h in gh:
            h.wait()
        wh.wait()

        @plsc.parallel_loop(0, _Q)
        def add_row(t, par=par):
            for w in range(_G // _WAVE):
                sls = [pl.ds((w * _WAVE + k) * 16, 16) for k in range(_WAVE)]
                wps = [wpe_b[par][t, sl] for sl in sls]
                toks = [[tok_v[par * _BATCH + b, t, sl] for sl in sls]
                        for b in range(_BATCH)]
                for b in range(_BATCH):
                    for k in range(_WAVE):
                        tok_v[par * _BATCH + b, t, sls[k]] = toks[b][k] + wps[k]

        store_h[q] = [pltpu.async_copy(
                          tok_v.at[par * _BATCH + b],
                          out_hbm.at[pl.ds(b * _BLOCK + pos_base + q * _Q, _Q)],
                          ss[par])
                      for b in range(_BATCH)]

    for q in (_NQ - 2, _NQ - 1):
        for h in store_h.pop(q):
            h.wait()


_emb_call = functools.partial(
    pl.kernel,
    out_type=jax.ShapeDtypeStruct((_BATCH * _BLOCK, _N_EMBD), jnp.float32),
    mesh=plsc.VectorSubcoreMesh(core_axis_name="c", subcore_axis_name="s"),
    scratch_types=[
        pltpu.VMEM((_BATCH * _P,), jnp.int32),
        pltpu.VMEM((_Q, _N_EMBD), jnp.float32),
        pltpu.VMEM((_Q, _N_EMBD), jnp.float32),
        pltpu.VMEM((2 * _BATCH, _Q, _N_EMBD), jnp.float32),
        pltpu.SemaphoreType.DMA,
        pltpu.SemaphoreType.DMA,
        pltpu.SemaphoreType.DMA,
        pltpu.SemaphoreType.DMA,
        pltpu.SemaphoreType.DMA,
        pltpu.SemaphoreType.DMA,
    ],
)(_emb_body)


def kernel(idx, tok_emb, wpe):
    idx_flat = idx.reshape(-1)
    out = _emb_call(idx_flat, tok_emb, wpe)
    return out.reshape(_BATCH, _BLOCK, _N_EMBD)
